# Initial kernel scaffold; baseline (speedup 1.0000x reference)
import jax
import jax.numpy as jnp
from jax.experimental import pallas as pl


def kernel(que_embeds, x_idx, edge_index, edge_attr, edge_type, ent_table, rel_init, W_ih, W_hh, b_ih, b_hh, mess_W, mess_b, atten_w, rel_W, rel_b, e_gamma, e_beta, r_gamma, r_beta):
    raise NotImplementedError("write your pallas kernel here")


# speedup vs baseline: 4.8610x; 4.8610x over previous
import functools

import jax
import jax.numpy as jnp
from jax import lax
from jax.experimental import pallas as pl
from jax.experimental.pallas import tpu as pltpu
from jax.experimental.pallas import tpu_sc as plsc

N_ENT = 100000
N_LOC = 10000
E = 320000
D = 128
NREL_PAD = 408
N_PAD = 10240
EPS = 1e-5

NC = 2
NS = 16
NW = NC * NS
E_PER_W = E // NW
EB = 80
NBLK = E_PER_W // EB
NACC = 10240
ROWS_PER_TILE = NACC // NS


def _lstm_body(que, wih, whh, b, c_out):
    def step(i, hc):
        h, c = hc
        xt = que[pl.ds(i, 1), :]
        gates = (
            lax.dot_general(xt, wih[...], (((1,), (1,)), ((), ())),
                            preferred_element_type=jnp.float32)
            + lax.dot_general(h, whh[...], (((1,), (1,)), ((), ())),
                              preferred_element_type=jnp.float32)
            + b[...]
        )
        ig = jax.nn.sigmoid(gates[:, 0:D])
        fg = jax.nn.sigmoid(gates[:, D:2 * D])
        gg = jnp.tanh(gates[:, 2 * D:3 * D])
        og = jax.nn.sigmoid(gates[:, 3 * D:4 * D])
        c2 = fg * c + ig * gg
        h2 = og * jnp.tanh(c2)
        return (h2, c2)

    zero = jnp.zeros((1, D), jnp.float32)
    _, c = lax.fori_loop(0, 16, step, (zero, zero))
    c_out[...] = c


def _run_lstm(que_embeds, W_ih, W_hh, b):
    return pl.pallas_call(
        _lstm_body,
        out_shape=jax.ShapeDtypeStruct((1, D), jnp.float32),
    )(que_embeds, W_ih, W_hh, b)


def _rel_body(rel_init, mess_W, mess_b, atten_w, c, z_out, t_out):
    w2 = mess_W[:, D:2 * D]
    rw = lax.dot_general(rel_init[...], w2, (((1,), (1,)), ((), ())),
                         preferred_element_type=jnp.float32)
    b = mess_b[...]
    z1 = rw + b
    z2 = -rw + b
    ztail = jnp.broadcast_to(b, (8, D))
    z = jnp.concatenate([z1, z2, ztail], axis=0)
    z_out[...] = z
    w_m = atten_w[:, 0:D]
    w_q = atten_w[:, D:2 * D]
    qcw = jnp.sum(c[...] * w_q)
    t_out[...] = jnp.sum(z * w_m, axis=1, keepdims=True) + qcw


def _run_rel(rel_init, mess_W, mess_b, atten_w, c):
    return pl.pallas_call(
        _rel_body,
        out_shape=[
            jax.ShapeDtypeStruct((NREL_PAD, D), jnp.float32),
            jax.ShapeDtypeStruct((NREL_PAD, 1), jnp.float32),
        ],
    )(rel_init, mess_W, mess_b, atten_w, c)


def _gather_rows_body(table_hbm, idx_hbm, out_hbm, idx_v, rows_v, sem):
    b_per_w = N_PAD // NW
    wid = lax.axis_index("c") * NS + lax.axis_index("s")
    base = wid * b_per_w
    pltpu.sync_copy(idx_hbm.at[pl.ds(base, b_per_w)], idx_v)
    pltpu.async_copy(table_hbm.at[idx_v], rows_v, sem).wait()
    pltpu.sync_copy(rows_v, out_hbm.at[pl.ds(base, b_per_w)])


def _run_gather_rows(table, idx_pad):
    b_per_w = N_PAD // NW
    mesh = plsc.VectorSubcoreMesh(core_axis_name="c", subcore_axis_name="s",
                                  num_cores=NC, num_subcores=NS)
    f = pl.kernel(
        _gather_rows_body,
        out_type=jax.ShapeDtypeStruct((N_PAD, D), jnp.float32),
        mesh=mesh,
        compiler_params=pltpu.CompilerParams(needs_layout_passes=False),
        scratch_types=[
            pltpu.VMEM((b_per_w,), jnp.int32),
            pltpu.VMEM((b_per_w, D), jnp.float32),
            pltpu.SemaphoreType.DMA,
        ],
    )
    return f(table, idx_pad)


def _node_body(xg, mess_W, atten_w, y_out, s_out):
    w1 = mess_W[:, 0:D]
    y = lax.dot_general(xg[...], w1, (((1,), (1,)), ((), ())),
                        preferred_element_type=jnp.float32)
    y_out[...] = y
    w_m = atten_w[:, 0:D]
    s_out[...] = jnp.sum(y * w_m, axis=1, keepdims=True)


def _run_node(xg, mess_W, atten_w):
    return pl.pallas_call(
        _node_body,
        out_shape=[
            jax.ShapeDtypeStruct((N_PAD, D), jnp.float32),
            jax.ShapeDtypeStruct((N_PAD, 1), jnp.float32),
        ],
    )(xg, mess_W, atten_w)


def _edge_body(src_h, dst_h, attr_h, y_h, z_h, s_h, t_h, zn_h, zd_h,
               num_o, den_o,
               s_v, t_v, src_v, dst_v, attr_v, y_rows, z_rows, a_v,
               num_sp, den_sp, sem_y, sem_z):
    cid = lax.axis_index("c")
    sid = lax.axis_index("s")
    wid = cid * NS + sid

    pltpu.sync_copy(s_h, s_v)
    pltpu.sync_copy(t_h, t_v)
    pltpu.sync_copy(zn_h.at[pl.ds(sid * ROWS_PER_TILE, ROWS_PER_TILE)],
                    num_sp.at[pl.ds(sid * ROWS_PER_TILE, ROWS_PER_TILE)])

    @pl.when(sid == 0)
    def _():
        pltpu.sync_copy(zd_h, den_sp)

    plsc.subcore_barrier()

    base_e = wid * E_PER_W

    def block(b, carry):
        off = base_e + b * EB
        pltpu.sync_copy(src_h.at[pl.ds(off, EB)], src_v)
        pltpu.sync_copy(dst_h.at[pl.ds(off, EB)], dst_v)
        pltpu.sync_copy(attr_h.at[pl.ds(off, EB)], attr_v)
        cp_y = pltpu.async_copy(y_h.at[src_v], y_rows, sem_y)
        cp_z = pltpu.async_copy(z_h.at[attr_v], z_rows, sem_z)

        def agrp(j, _):
            sv = src_v[pl.ds(j * 16, 16)]
            av = attr_v[pl.ds(j * 16, 16)]
            u = plsc.load_gather(s_v, [sv]) + plsc.load_gather(t_v, [av])
            au = jnp.abs(u)
            ex = jnp.exp(au * 2.0)
            th = 1.0 - 2.0 / (ex + 1.0)
            th = jnp.where(u < 0.0, -th, th)
            a_v[pl.ds(j * 16, 16)] = jnp.exp(th)
            return 0

        lax.fori_loop(0, EB // 16, agrp, 0, unroll=True)

        cp_y.wait()
        cp_z.wait()

        def edge(i, _):
            ab = plsc.load_gather(a_v, [jnp.broadcast_to(i, (16,)).astype(jnp.int32)])
            for cchunk in range(D // 16):
                sl = pl.ds(cchunk * 16, 16)
                y_rows[i, sl] = ab * (y_rows[i, sl] + z_rows[i, sl])
            return 0

        lax.fori_loop(0, EB, edge, 0)

        pltpu.sync_copy(y_rows, num_sp.at[dst_v], add=True)
        pltpu.sync_copy(a_v, den_sp.at[dst_v], add=True)
        return carry

    lax.fori_loop(0, NBLK, block, 0)

    plsc.subcore_barrier()

    pltpu.sync_copy(num_sp.at[pl.ds(sid * ROWS_PER_TILE, ROWS_PER_TILE)],
                    num_o.at[cid, pl.ds(sid * ROWS_PER_TILE, ROWS_PER_TILE)])

    @pl.when(sid == 0)
    def _():
        pltpu.sync_copy(den_sp, den_o.at[cid])


def _run_edges(src, dst, attr, y, z, s, t):
    mesh = plsc.VectorSubcoreMesh(core_axis_name="c", subcore_axis_name="s",
                                  num_cores=NC, num_subcores=NS)
    zn = jnp.zeros((NACC, D), jnp.float32)
    zd = jnp.zeros((NACC,), jnp.float32)
    f = pl.kernel(
        _edge_body,
        out_type=[
            jax.ShapeDtypeStruct((NC, NACC, D), jnp.float32),
            jax.ShapeDtypeStruct((NC, NACC), jnp.float32),
        ],
        mesh=mesh,
        compiler_params=pltpu.CompilerParams(needs_layout_passes=False),
        scratch_types=[
            pltpu.VMEM((N_PAD,), jnp.float32),
            pltpu.VMEM((NREL_PAD,), jnp.float32),
            pltpu.VMEM((EB,), jnp.int32),
            pltpu.VMEM((EB,), jnp.int32),
            pltpu.VMEM((EB,), jnp.int32),
            pltpu.VMEM((EB, D), jnp.float32),
            pltpu.VMEM((EB, D), jnp.float32),
            pltpu.VMEM((EB,), jnp.float32),
            pltpu.VMEM_SHARED((NACC, D), jnp.float32),
            pltpu.VMEM_SHARED((NACC,), jnp.float32),
            pltpu.SemaphoreType.DMA,
            pltpu.SemaphoreType.DMA,
        ],
    )
    return f(src, dst, attr, y, z, s, t, zn, zd)


def _final_body(num_p, den_p, c, gamma, beta, out_ref):
    num = (num_p[0] + num_p[1])[0:N_LOC]
    den = (den_p[0] + den_p[1])[0:N_LOC]
    pos = den > 0.0
    x = jnp.where(pos, num / jnp.where(pos, den, 1.0), 0.0)
    mu = jnp.mean(x, axis=0, keepdims=True)
    var = jnp.mean((x - mu) ** 2, axis=0, keepdims=True)
    x = gamma[...] * (x - mu) / jnp.sqrt(var + EPS) + beta[...]
    x = jnp.tanh(x)
    diff = c[...] - x
    dsq = jnp.sum(diff * diff, axis=1, keepdims=True)
    out_ref[...] = jax.nn.sigmoid(jnp.sqrt(dsq))


def _run_final(num_p, den_p, c, gamma, beta):
    return pl.pallas_call(
        _final_body,
        out_shape=jax.ShapeDtypeStruct((N_LOC, 1), jnp.float32),
    )(num_p, den_p, c, gamma, beta)


def kernel(que_embeds, x_idx, edge_index, edge_attr, edge_type,
           ent_table, rel_init, W_ih, W_hh, b_ih, b_hh,
           mess_W, mess_b, atten_w, rel_W, rel_b,
           e_gamma, e_beta, r_gamma, r_beta):
    b = (b_ih + b_hh).reshape(1, 4 * D)
    c = _run_lstm(que_embeds, W_ih, W_hh, b)
    z, t = _run_rel(rel_init, mess_W, mess_b.reshape(1, D), atten_w, c)
    idx_pad = jnp.concatenate(
        [x_idx.astype(jnp.int32), jnp.zeros((N_PAD - N_LOC,), jnp.int32)])
    xg = _run_gather_rows(ent_table, idx_pad)
    y, s = _run_node(xg, mess_W, atten_w)
    src = edge_index[0].astype(jnp.int32)
    dst = edge_index[1].astype(jnp.int32)
    attr = edge_attr.astype(jnp.int32)
    num_p, den_p = _run_edges(src, dst, attr, y, z,
                              s.reshape(N_PAD), t.reshape(NREL_PAD))
    out = _run_final(num_p, den_p.reshape(NC, NACC, 1), c,
                     e_gamma.reshape(1, D), e_beta.reshape(1, D))
    return out.reshape(N_LOC)


# speedup vs baseline: 5.7125x; 1.1752x over previous
import functools

import jax
import jax.numpy as jnp
from jax import lax
from jax.experimental import pallas as pl
from jax.experimental.pallas import tpu as pltpu
from jax.experimental.pallas import tpu_sc as plsc

N_ENT = 100000
N_LOC = 10000
E = 320000
D = 128
NREL_PAD = 408
N_PAD = 10240
EPS = 1e-5

NC = 2
NS = 16
NW = NC * NS
EB = 64
E_PER_W = 10240
E_PAD = NW * E_PER_W
NBLK = E_PER_W // EB
NPAIR = NBLK // 2
NACC = 10240
ROWS_PER_TILE = NACC // NS


def _lstm_body(que, wih, whh, b, c_out):
    def step(i, hc):
        h, c = hc
        xt = que[pl.ds(i, 1), :]
        gates = (
            lax.dot_general(xt, wih[...], (((1,), (1,)), ((), ())),
                            preferred_element_type=jnp.float32)
            + lax.dot_general(h, whh[...], (((1,), (1,)), ((), ())),
                              preferred_element_type=jnp.float32)
            + b[...]
        )
        ig = jax.nn.sigmoid(gates[:, 0:D])
        fg = jax.nn.sigmoid(gates[:, D:2 * D])
        gg = jnp.tanh(gates[:, 2 * D:3 * D])
        og = jax.nn.sigmoid(gates[:, 3 * D:4 * D])
        c2 = fg * c + ig * gg
        h2 = og * jnp.tanh(c2)
        return (h2, c2)

    zero = jnp.zeros((1, D), jnp.float32)
    _, c = lax.fori_loop(0, 16, step, (zero, zero))
    c_out[...] = c


def _run_lstm(que_embeds, W_ih, W_hh, b):
    return pl.pallas_call(
        _lstm_body,
        out_shape=jax.ShapeDtypeStruct((1, D), jnp.float32),
    )(que_embeds, W_ih, W_hh, b)


def _rel_body(rel_init, mess_W, mess_b, atten_w, c, z_out, t_out):
    w2 = mess_W[:, D:2 * D]
    rw = lax.dot_general(rel_init[...], w2, (((1,), (1,)), ((), ())),
                         preferred_element_type=jnp.float32)
    b = mess_b[...]
    z1 = rw + b
    z2 = -rw + b
    ztail = jnp.broadcast_to(b, (8, D))
    z = jnp.concatenate([z1, z2, ztail], axis=0)
    z_out[...] = z
    w_m = atten_w[:, 0:D]
    w_q = atten_w[:, D:2 * D]
    qcw = jnp.sum(c[...] * w_q)
    t_out[...] = jnp.sum(z * w_m, axis=1, keepdims=True) + qcw


def _run_rel(rel_init, mess_W, mess_b, atten_w, c):
    return pl.pallas_call(
        _rel_body,
        out_shape=[
            jax.ShapeDtypeStruct((NREL_PAD, D), jnp.float32),
            jax.ShapeDtypeStruct((NREL_PAD, 1), jnp.float32),
        ],
    )(rel_init, mess_W, mess_b, atten_w, c)


def _gather_rows_body(table_hbm, idx_hbm, out_hbm, idx_v, rows_v, sem):
    b_per_w = N_PAD // NW
    wid = lax.axis_index("c") * NS + lax.axis_index("s")
    base = wid * b_per_w
    pltpu.sync_copy(idx_hbm.at[pl.ds(base, b_per_w)], idx_v)
    pltpu.async_copy(table_hbm.at[idx_v], rows_v, sem).wait()
    pltpu.sync_copy(rows_v, out_hbm.at[pl.ds(base, b_per_w)])


def _run_gather_rows(table, idx_pad):
    b_per_w = N_PAD // NW
    mesh = plsc.VectorSubcoreMesh(core_axis_name="c", subcore_axis_name="s",
                                  num_cores=NC, num_subcores=NS)
    f = pl.kernel(
        _gather_rows_body,
        out_type=jax.ShapeDtypeStruct((N_PAD, D), jnp.float32),
        mesh=mesh,
        compiler_params=pltpu.CompilerParams(needs_layout_passes=False),
        scratch_types=[
            pltpu.VMEM((b_per_w,), jnp.int32),
            pltpu.VMEM((b_per_w, D), jnp.float32),
            pltpu.SemaphoreType.DMA,
        ],
    )
    return f(table, idx_pad)


def _node_body(xg, mess_W, atten_w, y_out, s_out):
    w1 = mess_W[:, 0:D]
    y = lax.dot_general(xg[...], w1, (((1,), (1,)), ((), ())),
                        preferred_element_type=jnp.float32)
    y_out[...] = y
    w_m = atten_w[:, 0:D]
    s_out[...] = jnp.sum(y * w_m, axis=1, keepdims=True)


def _run_node(xg, mess_W, atten_w):
    return pl.pallas_call(
        _node_body,
        out_shape=[
            jax.ShapeDtypeStruct((N_PAD, D), jnp.float32),
            jax.ShapeDtypeStruct((N_PAD, 1), jnp.float32),
        ],
    )(xg, mess_W, atten_w)


def _edge_body(src_h, dst_h, attr_h, y_h, z_h, s_h, t_h, zn_h, zd_h,
               num_o, den_o,
               s_v, t_v,
               sv0, dv0, av0, yb0, zb0, ab0,
               sv1, dv1, av1, yb1, zb1, ab1,
               num_sp, den_sp,
               si0, sy0, sz0, sn0, sd0,
               si1, sy1, sz1, sn1, sd1):
    cid = lax.axis_index("c")
    sid = lax.axis_index("s")
    wid = cid * NS + sid

    pltpu.sync_copy(s_h, s_v)
    pltpu.sync_copy(t_h, t_v)
    pltpu.sync_copy(zn_h.at[pl.ds(sid * ROWS_PER_TILE, ROWS_PER_TILE)],
                    num_sp.at[pl.ds(sid * ROWS_PER_TILE, ROWS_PER_TILE)])

    @pl.when(sid == 0)
    def _():
        pltpu.sync_copy(zd_h, den_sp)

    plsc.subcore_barrier()

    base_e = wid * E_PER_W
    A = (sv0, dv0, av0, yb0, zb0, ab0, si0, sy0, sz0, sn0, sd0)
    B = (sv1, dv1, av1, yb1, zb1, ab1, si1, sy1, sz1, sn1, sd1)

    def issue_idx(g, S):
        off = base_e + g * EB
        pltpu.async_copy(src_h.at[pl.ds(off, EB)], S[0], S[6])
        pltpu.async_copy(dst_h.at[pl.ds(off, EB)], S[1], S[6])
        pltpu.async_copy(attr_h.at[pl.ds(off, EB)], S[2], S[6])

    def wait_idx(S):
        for r in (S[0], S[1], S[2]):
            pltpu.make_async_copy(src_h.at[pl.ds(0, EB)], r, S[6]).wait()

    def issue_rows(S):
        pltpu.async_copy(y_h.at[S[0]], S[3], S[7])
        pltpu.async_copy(z_h.at[S[2]], S[4], S[8])

    def wait_rows(S):
        pltpu.make_async_copy(y_h.at[S[0]], S[3], S[7]).wait()
        pltpu.make_async_copy(z_h.at[S[2]], S[4], S[8]).wait()

    def compute_a(S):
        def agrp(j, _):
            svv = S[0][pl.ds(j * 16, 16)]
            avv = S[2][pl.ds(j * 16, 16)]
            u = plsc.load_gather(s_v, [svv]) + plsc.load_gather(t_v, [avv])
            au = jnp.abs(u)
            ex = jnp.exp(au * 2.0)
            th = 1.0 - 2.0 / (ex + 1.0)
            th = jnp.where(u < 0.0, -th, th)
            S[5][pl.ds(j * 16, 16)] = jnp.exp(th)
            return 0

        lax.fori_loop(0, EB // 16, agrp, 0, unroll=True)

    def scale(S):
        def edge(i, _):
            ab = plsc.load_gather(
                S[5], [jnp.broadcast_to(i, (16,)).astype(jnp.int32)])
            for cchunk in range(D // 16):
                sl = pl.ds(cchunk * 16, 16)
                S[3][i, sl] = ab * (S[3][i, sl] + S[4][i, sl])
            return 0

        lax.fori_loop(0, EB, edge, 0)

    def issue_scatter(S):
        pltpu.async_copy(S[3], num_sp.at[S[1]], S[9], add=True)
        pltpu.async_copy(S[5], den_sp.at[S[1]], S[10], add=True)

    def wait_scatter(S):
        pltpu.make_async_copy(S[3], num_sp.at[S[1]], S[9]).wait()
        pltpu.make_async_copy(S[5], den_sp.at[S[1]], S[10]).wait()

    issue_idx(0, A)
    wait_idx(A)
    issue_rows(A)

    def block(g, P, Q):
        compute_a(P)

        @pl.when(g >= 1)
        def _():
            wait_scatter(Q)

        @pl.when(g + 1 < NBLK)
        def _():
            issue_idx(g + 1, Q)

        wait_rows(P)

        @pl.when(g + 1 < NBLK)
        def _():
            wait_idx(Q)
            issue_rows(Q)

        scale(P)
        issue_scatter(P)

    def pairf(h, _):
        block(2 * h, A, B)
        block(2 * h + 1, B, A)
        return 0

    lax.fori_loop(0, NPAIR, pairf, 0)
    wait_scatter(B)

    plsc.subcore_barrier()

    pltpu.sync_copy(num_sp.at[pl.ds(sid * ROWS_PER_TILE, ROWS_PER_TILE)],
                    num_o.at[cid, pl.ds(sid * ROWS_PER_TILE, ROWS_PER_TILE)])

    @pl.when(sid == 0)
    def _():
        pltpu.sync_copy(den_sp, den_o.at[cid])


def _run_edges(src, dst, attr, y, z, s, t):
    mesh = plsc.VectorSubcoreMesh(core_axis_name="c", subcore_axis_name="s",
                                  num_cores=NC, num_subcores=NS)
    zn = jnp.zeros((NACC, D), jnp.float32)
    zd = jnp.zeros((NACC,), jnp.float32)
    f = pl.kernel(
        _edge_body,
        out_type=[
            jax.ShapeDtypeStruct((NC, NACC, D), jnp.float32),
            jax.ShapeDtypeStruct((NC, NACC), jnp.float32),
        ],
        mesh=mesh,
        compiler_params=pltpu.CompilerParams(needs_layout_passes=False),
        scratch_types=(
            [
                pltpu.VMEM((N_PAD,), jnp.float32),
                pltpu.VMEM((NREL_PAD,), jnp.float32),
            ]
            + 2 * [
                pltpu.VMEM((EB,), jnp.int32),
                pltpu.VMEM((EB,), jnp.int32),
                pltpu.VMEM((EB,), jnp.int32),
                pltpu.VMEM((EB, D), jnp.float32),
                pltpu.VMEM((EB, D), jnp.float32),
                pltpu.VMEM((EB,), jnp.float32),
            ]
            + [
                pltpu.VMEM_SHARED((NACC, D), jnp.float32),
                pltpu.VMEM_SHARED((NACC,), jnp.float32),
            ]
            + 10 * [pltpu.SemaphoreType.DMA]
        ),
    )
    return f(src, dst, attr, y, z, s, t, zn, zd)


def _final_body(num_p, den_p, c, gamma, beta, out_ref):
    num = (num_p[0] + num_p[1])[0:N_LOC]
    den = (den_p[0] + den_p[1])[0:N_LOC]
    pos = den > 0.0
    x = jnp.where(pos, num / jnp.where(pos, den, 1.0), 0.0)
    mu = jnp.mean(x, axis=0, keepdims=True)
    var = jnp.mean((x - mu) ** 2, axis=0, keepdims=True)
    x = gamma[...] * (x - mu) / jnp.sqrt(var + EPS) + beta[...]
    x = jnp.tanh(x)
    diff = c[...] - x
    dsq = jnp.sum(diff * diff, axis=1, keepdims=True)
    out_ref[...] = jax.nn.sigmoid(jnp.sqrt(dsq))


def _run_final(num_p, den_p, c, gamma, beta):
    return pl.pallas_call(
        _final_body,
        out_shape=jax.ShapeDtypeStruct((N_LOC, 1), jnp.float32),
    )(num_p, den_p, c, gamma, beta)


def kernel(que_embeds, x_idx, edge_index, edge_attr, edge_type,
           ent_table, rel_init, W_ih, W_hh, b_ih, b_hh,
           mess_W, mess_b, atten_w, rel_W, rel_b,
           e_gamma, e_beta, r_gamma, r_beta):
    b = (b_ih + b_hh).reshape(1, 4 * D)
    c = _run_lstm(que_embeds, W_ih, W_hh, b)
    z, t = _run_rel(rel_init, mess_W, mess_b.reshape(1, D), atten_w, c)
    idx_pad = jnp.concatenate(
        [x_idx.astype(jnp.int32), jnp.zeros((N_PAD - N_LOC,), jnp.int32)])
    xg = _run_gather_rows(ent_table, idx_pad)
    y, s = _run_node(xg, mess_W, atten_w)
    pad_n = E_PAD - E
    src = jnp.concatenate(
        [edge_index[0].astype(jnp.int32), jnp.zeros((pad_n,), jnp.int32)])
    dst = jnp.concatenate(
        [edge_index[1].astype(jnp.int32),
         N_LOC + (jnp.arange(pad_n, dtype=jnp.int32) % (NACC - N_LOC))])
    attr = jnp.concatenate(
        [edge_attr.astype(jnp.int32), jnp.zeros((pad_n,), jnp.int32)])
    num_p, den_p = _run_edges(src, dst, attr, y, z,
                              s.reshape(N_PAD), t.reshape(NREL_PAD))
    out = _run_final(num_p, den_p.reshape(NC, NACC, 1), c,
                     e_gamma.reshape(1, D), e_beta.reshape(1, D))
    return out.reshape(N_LOC)
